# baseline (speedup 1.0000x reference)
import jax
import jax.numpy as jnp
from jax.experimental import pallas as pl

N = 256
NIN = 50176
NOUT = 1024
COUT = 4
NCLS = 10
D = 16

SUB = 3584
NSUB = 7
CHUNK = SUB * NSUB
NCHUNKS = NIN // CHUNK


def _agg_body(x_ref, ids_ref, vt_ref, out_ref):
    i = pl.program_id(0)
    part = jnp.zeros((N, D), jnp.float32)
    vt = vt_ref[...]
    for s in range(NSUB):
        ids = ids_ref[0, 0, pl.ds(s * SUB, SUB)]
        ids_b = jnp.broadcast_to(ids.reshape(1, SUB), (D, SUB))
        gt = jnp.zeros((D, SUB), jnp.float32)
        for t in range(NOUT // 128):
            local = ids_b - (t * 128)
            inb = (local >= 0) & (local < 128)
            safe = jnp.where(inb, local, 0)
            got = jnp.take_along_axis(vt[:, t * 128:(t + 1) * 128],
                                      safe, axis=1)
            gt = jnp.where(inb, got, gt)
        part += jax.lax.dot_general(
            x_ref[:, pl.ds(s * SUB, SUB)], gt,
            (((1,), (1,)), ((), ())),
            preferred_element_type=jnp.float32)

    @pl.when(i == 0)
    def _init():
        out_ref[...] = part

    @pl.when(i > 0)
    def _acc():
        out_ref[...] += part


def kernel(x, region_ids, W, b, fc_w, fc_b):
    fcr = fc_w.reshape(COUT, NOUT, NCLS)
    v = jnp.einsum('jo,ojc->jc', W[:, 0, :], fcr)
    const = jnp.einsum('jo,ojc->c', b, fcr) + fc_b
    vt = jnp.pad(v, ((0, 0), (0, D - NCLS))).T
    ids2 = region_ids.reshape(NCHUNKS, 1, CHUNK)

    out_pad = pl.pallas_call(
        _agg_body,
        grid=(NCHUNKS,),
        in_specs=[
            pl.BlockSpec((N, CHUNK), lambda i: (0, i)),
            pl.BlockSpec((1, 1, CHUNK), lambda i: (i, 0, 0)),
            pl.BlockSpec((D, NOUT), lambda i: (0, 0)),
        ],
        out_specs=pl.BlockSpec((N, D), lambda i: (0, 0)),
        out_shape=jax.ShapeDtypeStruct((N, D), jnp.float32),
    )(x, ids2, vt)

    return out_pad[:, :NCLS] + const


# speedup vs baseline: 1.0482x; 1.0482x over previous
import jax
import jax.numpy as jnp
from jax.experimental import pallas as pl

N = 256
NIN = 50176
NOUT = 1024
COUT = 4
NCLS = 10
D = 16

SUB = 1792
NSUB = 4
CHUNK = SUB * NSUB
NCHUNKS = NIN // CHUNK


def _agg_body(x_ref, ids_ref, vt_ref, out_ref):
    i = pl.program_id(0)
    part = jnp.zeros((N, D), jnp.float32)
    vt = vt_ref[...]
    for s in range(NSUB):
        ids = ids_ref[0, 0, pl.ds(s * SUB, SUB)]
        ids_b = jnp.broadcast_to(ids.reshape(1, SUB), (D, SUB))
        gt = jnp.zeros((D, SUB), jnp.float32)
        for t in range(NOUT // 128):
            local = ids_b - (t * 128)
            inb = (local >= 0) & (local < 128)
            safe = jnp.where(inb, local, 0)
            got = jnp.take_along_axis(vt[:, t * 128:(t + 1) * 128],
                                      safe, axis=1)
            gt = jnp.where(inb, got, gt)
        part += jax.lax.dot_general(
            x_ref[:, pl.ds(s * SUB, SUB)], gt,
            (((1,), (1,)), ((), ())),
            preferred_element_type=jnp.float32)

    @pl.when(i == 0)
    def _init():
        out_ref[...] = part

    @pl.when(i > 0)
    def _acc():
        out_ref[...] += part


def kernel(x, region_ids, W, b, fc_w, fc_b):
    fcr = fc_w.reshape(COUT, NOUT, NCLS)
    v = jnp.einsum('jo,ojc->jc', W[:, 0, :], fcr)
    const = jnp.einsum('jo,ojc->c', b, fcr) + fc_b
    vt = jnp.pad(v, ((0, 0), (0, D - NCLS))).T
    ids2 = region_ids.reshape(NCHUNKS, 1, CHUNK)

    out_pad = pl.pallas_call(
        _agg_body,
        grid=(NCHUNKS,),
        in_specs=[
            pl.BlockSpec((N, CHUNK), lambda i: (0, i)),
            pl.BlockSpec((1, 1, CHUNK), lambda i: (i, 0, 0)),
            pl.BlockSpec((D, NOUT), lambda i: (0, 0)),
        ],
        out_specs=pl.BlockSpec((N, D), lambda i: (0, 0)),
        out_shape=jax.ShapeDtypeStruct((N, D), jnp.float32),
    )(x, ids2, vt)

    return out_pad[:, :NCLS] + const


# speedup vs baseline: 1.0851x; 1.0352x over previous
import jax
import jax.numpy as jnp
from jax.experimental import pallas as pl

N = 256
NIN = 50176
NOUT = 1024
COUT = 4
NCLS = 10
D = 16

SUB = 896
NSUB = 14
CHUNK = SUB * NSUB
NCHUNKS = NIN // CHUNK


def _agg_body(x_ref, ids_ref, vt_ref, out_ref):
    i = pl.program_id(0)
    part = jnp.zeros((N, D), jnp.float32)
    vt = vt_ref[...]
    for s in range(NSUB):
        ids = ids_ref[0, 0, pl.ds(s * SUB, SUB)]
        ids_b = jnp.broadcast_to(ids.reshape(1, SUB), (D, SUB))
        gt = jnp.zeros((D, SUB), jnp.float32)
        for t in range(NOUT // 128):
            local = ids_b - (t * 128)
            inb = (local >= 0) & (local < 128)
            safe = jnp.where(inb, local, 0)
            got = jnp.take_along_axis(vt[:, t * 128:(t + 1) * 128],
                                      safe, axis=1)
            gt = jnp.where(inb, got, gt)
        part += jax.lax.dot_general(
            x_ref[:, pl.ds(s * SUB, SUB)], gt,
            (((1,), (1,)), ((), ())),
            preferred_element_type=jnp.float32)

    @pl.when(i == 0)
    def _init():
        out_ref[...] = part

    @pl.when(i > 0)
    def _acc():
        out_ref[...] += part


def kernel(x, region_ids, W, b, fc_w, fc_b):
    fcr = fc_w.reshape(COUT, NOUT, NCLS)
    v = jnp.einsum('jo,ojc->jc', W[:, 0, :], fcr)
    const = jnp.einsum('jo,ojc->c', b, fcr) + fc_b
    vt = jnp.pad(v, ((0, 0), (0, D - NCLS))).T
    ids2 = region_ids.reshape(NCHUNKS, 1, CHUNK)

    out_pad = pl.pallas_call(
        _agg_body,
        grid=(NCHUNKS,),
        in_specs=[
            pl.BlockSpec((N, CHUNK), lambda i: (0, i)),
            pl.BlockSpec((1, 1, CHUNK), lambda i: (i, 0, 0)),
            pl.BlockSpec((D, NOUT), lambda i: (0, 0)),
        ],
        out_specs=pl.BlockSpec((N, D), lambda i: (0, 0)),
        out_shape=jax.ShapeDtypeStruct((N, D), jnp.float32),
    )(x, ids2, vt)

    return out_pad[:, :NCLS] + const


# speedup vs baseline: 1.1011x; 1.0148x over previous
import jax
import jax.numpy as jnp
from jax.experimental import pallas as pl

N = 256
NIN = 50176
NOUT = 1024
COUT = 4
NCLS = 10
D = 16

SUB = 896
NSUB = 14
CHUNK = SUB * NSUB
NCHUNKS = NIN // CHUNK
TBL = 128


def _agg_body(x_ref, ids_ref, vt_ref, out_ref):
    i = pl.program_id(0)
    part = jnp.zeros((N, D), jnp.float32)
    vt = vt_ref[...]
    for s in range(NSUB):
        ids = ids_ref[0, 0, pl.ds(s * SUB, SUB)]
        ids_b = jnp.broadcast_to(ids.reshape(1, SUB), (D, SUB))
        gt = jnp.zeros((D, SUB), jnp.float32)
        for t in range(NOUT // TBL):
            local = ids_b - (t * TBL)
            inb = (local >= 0) & (local < TBL)
            safe = jnp.where(inb, local, 0)
            got = jnp.take_along_axis(vt[:, t * TBL:(t + 1) * TBL],
                                      safe, axis=1)
            gt = jnp.where(inb, got, gt)
        part += jax.lax.dot_general(
            x_ref[:, pl.ds(s * SUB, SUB)], gt,
            (((1,), (1,)), ((), ())),
            preferred_element_type=jnp.float32)

    @pl.when(i == 0)
    def _init():
        out_ref[...] = part

    @pl.when(i > 0)
    def _acc():
        out_ref[...] += part


def kernel(x, region_ids, W, b, fc_w, fc_b):
    fcr = fc_w.reshape(COUT, NOUT, NCLS)
    v = jnp.einsum('jo,ojc->jc', W[:, 0, :], fcr)
    const = jnp.einsum('jo,ojc->c', b, fcr) + fc_b
    vt = jnp.pad(v, ((0, 0), (0, D - NCLS))).T
    ids3 = region_ids.reshape(NCHUNKS, 1, CHUNK)

    out_pad = pl.pallas_call(
        _agg_body,
        grid=(NCHUNKS,),
        in_specs=[
            pl.BlockSpec((N, CHUNK), lambda i: (0, i)),
            pl.BlockSpec((1, 1, CHUNK), lambda i: (i, 0, 0)),
            pl.BlockSpec((D, NOUT), lambda i: (0, 0)),
        ],
        out_specs=pl.BlockSpec((N, D), lambda i: (0, 0)),
        out_shape=jax.ShapeDtypeStruct((N, D), jnp.float32),
    )(x, ids3, vt)

    return out_pad[:, :NCLS] + const
